# baseline (speedup 1.0000x reference)
import functools

import jax
import jax.numpy as jnp
from jax import lax
from jax.experimental import pallas as pl
from jax.experimental.pallas import tpu as pltpu
from jax.experimental.pallas import tpu_sc as plsc

V = 1000000
D = 16
B = 4096
L = 50
FC1_OUT = 128
NUM_CLASSES = 10

NC, NS = 2, 16
NW = NC * NS
BPW = B // NW
CH = 64
NCHUNK = BPW // CH
G = 128
EL = CH * L
NG = EL // G


LP = 64


def _sc_pool(idx0, idx1, idx2, idx3, w_pad, E0, E1, E2, E3):
    mesh = plsc.VectorSubcoreMesh(core_axis_name="c", subcore_axis_name="s")

    @functools.partial(
        pl.kernel,
        out_type=jax.ShapeDtypeStruct((4, B, D), jnp.float32),
        mesh=mesh,
        scratch_types=[
            pltpu.VMEM((EL,), jnp.int32),
            pltpu.VMEM((CH, LP), jnp.float32),
            pltpu.VMEM((EL, D), jnp.float32),
            pltpu.VMEM((CH, D), jnp.float32),
            pltpu.SemaphoreType.DMA,
        ],
        compiler_params=pltpu.CompilerParams(use_tc_tiling_on_sc=False),
    )
    def k(i0, i1, i2, i3, w_hbm, t0, t1, t2, t3, out_hbm,
          idx_v, w_v, rows_v, out_v, sem):
        wid = lax.axis_index("s") * NC + lax.axis_index("c")

        for f, (ix, tab) in enumerate(((i0, t0), (i1, t1), (i2, t2), (i3, t3))):

            def chunk_body(c, carry, ix=ix, f=f, tab=tab):
                b0 = wid * BPW + c * CH
                pltpu.sync_copy(ix.at[pl.ds(b0 * L, EL)], idx_v)
                pltpu.sync_copy(w_hbm.at[pl.ds(b0, CH)], w_v)
                cps = [
                    pltpu.async_copy(
                        tab.at[idx_v.at[pl.ds(g * G, G)]],
                        rows_v.at[pl.ds(g * G, G)],
                        sem,
                    )
                    for g in range(NG)
                ]
                for cp in cps:
                    cp.wait()

                def row_body(jj, rc):
                    e0 = jj * L
                    wv = [w_v[jj, pl.ds(16 * kk, 16)] for kk in range(4)]
                    accs = [wv[0][i] * rows_v[e0 + i] for i in range(4)]
                    for l in range(4, L):
                        accs[l % 4] = (
                            accs[l % 4] + wv[l // 16][l % 16] * rows_v[e0 + l]
                        )
                    out_v[jj] = (accs[0] + accs[1]) + (accs[2] + accs[3])
                    return rc

                lax.fori_loop(0, CH, row_body, 0)
                pltpu.sync_copy(out_v, out_hbm.at[f, pl.ds(b0, CH)])
                return carry

            lax.fori_loop(0, NCHUNK, chunk_body, 0)

    return k(idx0, idx1, idx2, idx3, w_pad, E0, E1, E2, E3)


def _mlp_body(p_ref, w1_ref, b1_ref, w2_ref, b2_ref, o_ref):
    h = jnp.zeros((p_ref.shape[1], FC1_OUT), jnp.float32)
    for f in range(4):
        h = h + jnp.dot(p_ref[f], w1_ref[f], preferred_element_type=jnp.float32)
    h = jnp.maximum(h + b1_ref[0], 0.0)
    o = jnp.dot(h, w2_ref[...], preferred_element_type=jnp.float32) + b2_ref[0]
    o_ref[...] = o


def _mlp(pooled, W1, b1, W2, b2):
    BLK = 512
    grid = (B // BLK,)
    return pl.pallas_call(
        _mlp_body,
        grid=grid,
        in_specs=[
            pl.BlockSpec((4, BLK, D), lambda i: (0, i, 0)),
            pl.BlockSpec((4, D, FC1_OUT), lambda i: (0, 0, 0)),
            pl.BlockSpec((1, FC1_OUT), lambda i: (0, 0)),
            pl.BlockSpec((FC1_OUT, NUM_CLASSES), lambda i: (0, 0)),
            pl.BlockSpec((1, NUM_CLASSES), lambda i: (0, 0)),
        ],
        out_specs=pl.BlockSpec((BLK, NUM_CLASSES), lambda i: (i, 0)),
        out_shape=jax.ShapeDtypeStruct((B, NUM_CLASSES), jnp.float32),
    )(pooled, W1.reshape(4, D, FC1_OUT), b1.reshape(1, FC1_OUT), W2,
      b2.reshape(1, NUM_CLASSES))


def kernel(x, E0, E1, E2, E3, W1, b1, W2, b2):
    idx = [x[f].reshape(B * L) for f in range(4)]
    w_pad = jnp.pad(x[4].astype(jnp.float32), ((0, 0), (0, LP - L)))
    pooled = _sc_pool(idx[0], idx[1], idx[2], idx[3], w_pad, E0, E1, E2, E3)
    return _mlp(pooled, W1, b1, W2, b2)
